# Initial kernel scaffold; baseline (speedup 1.0000x reference)
import jax
import jax.numpy as jnp
from jax.experimental import pallas as pl


def kernel(x, edge_attr, params, edge_index, batch):
    raise NotImplementedError("write your pallas kernel here")


# speedup vs baseline: 1.0691x; 1.0691x over previous
import functools
import jax
import jax.numpy as jnp
from jax import lax
from jax.experimental import pallas as pl
from jax.experimental.pallas import tpu as pltpu

_BN = 256


def _mm_kernel(x_ref, w_ref, b_ref, o_ref, *, relu):
    acc = jnp.dot(x_ref[...], w_ref[...], preferred_element_type=jnp.float32)
    acc = acc + b_ref[...]
    if relu:
        acc = jnp.maximum(acc, 0.0)
    o_ref[...] = acc


def _mm(x, W, b, relu):
    n = x.shape[0]
    d_in = x.shape[1]
    d_out = W.shape[1]
    npad = ((n + _BN - 1) // _BN) * _BN
    if npad != n:
        x = jnp.pad(x, ((0, npad - n), (0, 0)))
    out = pl.pallas_call(
        functools.partial(_mm_kernel, relu=relu),
        grid=(npad // _BN,),
        in_specs=[
            pl.BlockSpec((_BN, d_in), lambda i: (i, 0)),
            pl.BlockSpec((d_in, d_out), lambda i: (0, 0)),
            pl.BlockSpec((1, d_out), lambda i: (0, 0)),
        ],
        out_specs=pl.BlockSpec((_BN, d_out), lambda i: (i, 0)),
        out_shape=jax.ShapeDtypeStruct((npad, d_out), jnp.float32),
    )(x, W, b.reshape(1, d_out))
    return out[:n]


def _gcn(h, src, dst, ea, em, W, b, We):
    n = h.shape[0]
    deg = jnp.zeros((n,), h.dtype).at[dst].add(em) + 1.0
    norm = lax.rsqrt(deg[src] * deg[dst])
    msg = (h[src] * norm[:, None] + ea @ We) * em[:, None]
    agg = jnp.zeros_like(h).at[dst].add(msg)
    return _mm(agg, W, b, relu=True)


def _pool(h, src, dst, em, bat, p, k):
    n = h.shape[0]
    score = (h @ p) / (jnp.linalg.norm(p) + 1e-8)
    topv, perm = lax.top_k(score, k)
    hn = h[perm] * jax.nn.sigmoid(topv)[:, None]
    keep = jnp.zeros((n,), bool).at[perm].set(True)
    inv = jnp.zeros((n,), src.dtype).at[perm].set(jnp.arange(k, dtype=src.dtype))
    em2 = em * keep[src].astype(em.dtype) * keep[dst].astype(em.dtype)
    return hn, inv[src], inv[dst], em2, bat[perm], perm


def _segment_mean(h, gi, g):
    s = jnp.zeros((g, h.shape[1]), h.dtype).at[gi].add(h)
    c = jnp.zeros((g,), h.dtype).at[gi].add(1.0)
    return s / jnp.maximum(c, 1.0)[:, None]


def kernel(x, edge_attr, params, edge_index, batch):
    P = params
    g = 8
    ratios = [0.5, 0.5]
    pool_num = 2

    feat = _mm(x, P['W_enc'], P['b_enc'], relu=False)
    e = edge_index.shape[1]
    em = jnp.ones((e,), x.dtype)
    src0, dst0 = edge_index[0], edge_index[1]
    h = _gcn(feat, src0, dst0, edge_attr, em, P['W_in'], P['b_in'], P['We_in'])

    src, dst, bat = src0, dst0, batch
    down, levels, perms, gis = [], [], [], []
    for i in range(pool_num):
        h = _gcn(h, src, dst, edge_attr, em,
                 P['W_d%d' % i], P['b_d%d' % i], P['We_d%d' % i])
        levels.append((src, dst, em))
        gis.append(bat)
        down.append(h)
        k = int(h.shape[0] * ratios[i])
        h, src, dst, em, bat, perm = _pool(h, src, dst, em, bat, P['p%d' % i], k)
        perms.append(perm)

    h = _gcn(h, src, dst, edge_attr, em, P['W_b'], P['b_b'], P['We_b'])

    hs, hs_gis = [], []
    for i in range(pool_num):
        up = pool_num - 1 - i
        src, dst, em = levels[up]
        h = jnp.zeros((down[up].shape[0], h.shape[1]), h.dtype).at[perms[up]].set(h)
        h = _gcn(h, src, dst, edge_attr, em,
                 P['W_u%d' % i], P['b_u%d' % i], P['We_u%d' % i])
        h = h + down[up]
        hs.append(h)
        hs_gis.append(gis[up])
    h = h + feat
    hs.append(h)
    hs_gis.append(gis[0])

    rep = jnp.zeros((g, h.shape[1]), x.dtype)
    for hh, gi in zip(hs, hs_gis):
        rep = rep + _segment_mean(hh, gi, g)
    return _mm(rep, P['W_pred'], P['b_pred'], relu=False)


# speedup vs baseline: 1.1125x; 1.0406x over previous
import functools
import jax
import jax.numpy as jnp
from jax import lax
from jax.experimental import pallas as pl
from jax.experimental.pallas import tpu as pltpu
from jax.experimental.pallas import tpu_sc as plsc

_BN = 256
_CH = 128
_NW = 32


def _ceil_to(x, m):
    return ((x + m - 1) // m) * m


def _mm_kernel(x_ref, w_ref, b_ref, o_ref, *, relu):
    acc = jnp.dot(x_ref[...], w_ref[...], preferred_element_type=jnp.float32)
    acc = acc + b_ref[...]
    if relu:
        acc = jnp.maximum(acc, 0.0)
    o_ref[...] = acc


def _mm(x, W, b, relu):
    n = x.shape[0]
    d_in = x.shape[1]
    d_out = W.shape[1]
    npad = _ceil_to(n, _BN)
    if npad != n:
        x = jnp.pad(x, ((0, npad - n), (0, 0)))
    out = pl.pallas_call(
        functools.partial(_mm_kernel, relu=relu),
        grid=(npad // _BN,),
        in_specs=[
            pl.BlockSpec((_BN, d_in), lambda i: (i, 0)),
            pl.BlockSpec((d_in, d_out), lambda i: (0, 0)),
            pl.BlockSpec((1, d_out), lambda i: (0, 0)),
        ],
        out_specs=pl.BlockSpec((_BN, d_out), lambda i: (i, 0)),
        out_shape=jax.ShapeDtypeStruct((npad, d_out), jnp.float32),
    )(x, W, b.reshape(1, d_out))
    return out[:n]


def _epi_kernel(g0_ref, g1_ref, a_ref, we_ref, w_ref, b_ref, s_ref, o_ref, *,
                has_skip):
    r = lax.rsqrt(1.0 + a_ref[:, 4:5])
    m = (g0_ref[...] + g1_ref[...]) * r
    m = m + jnp.dot(a_ref[...], we_ref[...], preferred_element_type=jnp.float32)
    o = jnp.dot(m, w_ref[...], preferred_element_type=jnp.float32) + b_ref[...]
    o = jnp.maximum(o, 0.0)
    if has_skip:
        o = o + s_ref[...]
    o_ref[...] = o


def _conv_epilogue(gpair, a16, We, W, b, n, skip=None):
    nacc = gpair.shape[1]
    we16 = jnp.zeros((16, We.shape[1]), jnp.float32).at[:4].set(We)
    has_skip = skip is not None
    if has_skip:
        skip_p = jnp.pad(skip, ((0, nacc - skip.shape[0]), (0, 0)))
    else:
        skip_p = jnp.zeros((_BN, 128), jnp.float32)
    sspec = (pl.BlockSpec((_BN, 128), lambda i: (i, 0)) if has_skip
             else pl.BlockSpec((_BN, 128), lambda i: (0, 0)))
    out = pl.pallas_call(
        functools.partial(_epi_kernel, has_skip=has_skip),
        grid=(nacc // _BN,),
        in_specs=[
            pl.BlockSpec((_BN, 128), lambda i: (i, 0)),
            pl.BlockSpec((_BN, 128), lambda i: (i, 0)),
            pl.BlockSpec((_BN, 16), lambda i: (i, 0)),
            pl.BlockSpec((16, 128), lambda i: (0, 0)),
            pl.BlockSpec((128, 128), lambda i: (0, 0)),
            pl.BlockSpec((1, 128), lambda i: (0, 0)),
            sspec,
        ],
        out_specs=pl.BlockSpec((_BN, 128), lambda i: (i, 0)),
        out_shape=jax.ShapeDtypeStruct((nacc, 128), jnp.float32),
    )(gpair[0], gpair[1], a16, we16, W, b.reshape(1, 128), skip_p)
    return out[:n]


def _sc_edge_op(nsrc, nacc, epad, d, gather):
    ew = epad // _NW
    nch = ew // _CH
    stripe = nacc // 16
    zc = stripe // _CH
    mesh = plsc.VectorSubcoreMesh(core_axis_name="c", subcore_axis_name="s")

    def body(*refs):
        if gather:
            h_hbm, src_hbm, dst_hbm, z_hbm, out_hbm, sidx, didx, rows, acc, sem = refs
        else:
            vals_hbm, dst_hbm, z_hbm, out_hbm, didx, rows, acc, sem = refs
        cid = lax.axis_index("c")
        sid = lax.axis_index("s")
        wid = sid * 2 + cid
        base0 = sid * stripe
        pltpu.sync_copy(z_hbm, rows)
        for j in range(zc):
            pltpu.sync_copy(rows, acc.at[pl.ds(base0 + j * _CH, _CH)])
        plsc.subcore_barrier()

        def step(i, carry):
            base = wid * ew + i * _CH
            pltpu.sync_copy(dst_hbm.at[pl.ds(base, _CH)], didx)
            if gather:
                pltpu.sync_copy(src_hbm.at[pl.ds(base, _CH)], sidx)
                pltpu.async_copy(h_hbm.at[sidx], rows, sem).wait()
            else:
                pltpu.sync_copy(vals_hbm.at[pl.ds(base, _CH)], rows)
            pltpu.sync_copy(rows, acc.at[didx], add=True)
            return carry

        lax.fori_loop(0, nch, step, 0)
        plsc.subcore_barrier()
        for j in range(zc):
            pltpu.sync_copy(acc.at[pl.ds(base0 + j * _CH, _CH)], rows)
            pltpu.sync_copy(rows, out_hbm.at[cid, pl.ds(base0 + j * _CH, _CH)])

    scratch = []
    if gather:
        scratch.append(pltpu.VMEM((_CH,), jnp.int32))
    scratch += [
        pltpu.VMEM((_CH,), jnp.int32),
        pltpu.VMEM((_CH, d), jnp.float32),
        pltpu.VMEM_SHARED((nacc, d), jnp.float32),
        pltpu.SemaphoreType.DMA,
    ]
    return pl.kernel(
        body,
        out_type=jax.ShapeDtypeStruct((2, nacc, d), jnp.float32),
        mesh=mesh,
        scratch_types=scratch,
    )


def _sc_gather_scatter(h, src_eff, dst_eff, nacc):
    n = h.shape[0]
    nsrc = _ceil_to(n + 1, 8)
    hp = jnp.pad(h, ((0, nsrc - n), (0, 0)))
    k = _sc_edge_op(nsrc, nacc, src_eff.shape[0], 128, gather=True)
    return k(hp, src_eff, dst_eff, jnp.zeros((_CH, 128), jnp.float32))


def _sc_attr_deg(vals, dst_eff, nacc):
    k = _sc_edge_op(0, nacc, dst_eff.shape[0], 16, gather=False)
    return k(vals, dst_eff, jnp.zeros((_CH, 16), jnp.float32))


def _segment_mean(h, gi, g):
    s = jnp.zeros((g, h.shape[1]), h.dtype).at[gi].add(h)
    c = jnp.zeros((g,), h.dtype).at[gi].add(1.0)
    return s / jnp.maximum(c, 1.0)[:, None]


def kernel(x, edge_attr, params, edge_index, batch):
    P = params
    g = 8
    n0 = x.shape[0]
    e = edge_index.shape[1]
    k1, k2 = n0 // 2, n0 // 4
    nacc0, nacc1, nacc2 = (_ceil_to(n0 + 1, 2048), _ceil_to(k1 + 1, 2048),
                           _ceil_to(k2 + 1, 2048))
    epad = _ceil_to(e, _NW * _CH)

    src0 = edge_index[0]
    dst0 = edge_index[1]
    pad_e = epad - e

    def padi(a, fill):
        return jnp.pad(a, (0, pad_e), constant_values=fill).astype(jnp.int32)

    vals = jnp.zeros((epad, 16), jnp.float32)
    vals = vals.at[:e, :4].set(edge_attr).at[:e, 4].set(1.0)

    dst_l0 = padi(dst0, n0)
    a16_0p = _sc_attr_deg(vals, dst_l0, nacc0)
    a16_0 = a16_0p[0] + a16_0p[1]
    r0 = lax.rsqrt(1.0 + a16_0[:n0, 4])

    feat = _mm(x, P['W_enc'], P['b_enc'], relu=False)

    src_l0 = padi(src0, n0)
    h_in = _conv_epilogue(
        _sc_gather_scatter(feat * r0[:, None], src_l0, dst_l0, nacc0),
        a16_0, P['We_in'], P['W_in'], P['b_in'], n0)

    h_d0 = _conv_epilogue(
        _sc_gather_scatter(h_in * r0[:, None], src_l0, dst_l0, nacc0),
        a16_0, P['We_d0'], P['W_d0'], P['b_d0'], n0)

    p0 = P['p0']
    pm0 = jnp.zeros((128, 128), jnp.float32).at[:, 0].set(p0)
    score0 = _mm(h_d0, pm0, jnp.zeros((128,), jnp.float32), relu=False)[:, 0]
    score0 = score0 / (jnp.linalg.norm(p0) + 1e-8)
    topv0, perm0 = lax.top_k(score0, k1)
    sig0 = jax.nn.sigmoid(topv0)
    keep0 = jnp.zeros((n0,), bool).at[perm0].set(True)
    inv0 = jnp.zeros((n0,), jnp.int32).at[perm0].set(
        jnp.arange(k1, dtype=jnp.int32))
    em1 = keep0[src0] & keep0[dst0]
    bat1 = batch[perm0]

    dst_l1 = padi(jnp.where(em1, inv0[dst0], k1), k1)
    a16_1p = _sc_attr_deg(vals, dst_l1, nacc1)
    a16_1 = a16_1p[0] + a16_1p[1]
    r1 = lax.rsqrt(1.0 + a16_1[:k1, 4])

    sfull1 = jnp.zeros((n0,), jnp.float32).at[perm0].set(sig0 * r1)
    h_d1 = _conv_epilogue(
        _sc_gather_scatter(h_d0 * sfull1[:, None], src_l0, dst_l1, nacc1),
        a16_1, P['We_d1'], P['W_d1'], P['b_d1'], k1)

    p1 = P['p1']
    pm1 = jnp.zeros((128, 128), jnp.float32).at[:, 0].set(p1)
    score1 = _mm(h_d1, pm1, jnp.zeros((128,), jnp.float32), relu=False)[:, 0]
    score1 = score1 / (jnp.linalg.norm(p1) + 1e-8)
    topv1, perm1 = lax.top_k(score1, k2)
    sig1 = jax.nn.sigmoid(topv1)
    src1 = inv0[src0]
    dst1 = inv0[dst0]
    keep1 = jnp.zeros((k1,), bool).at[perm1].set(True)
    inv1 = jnp.zeros((k1,), jnp.int32).at[perm1].set(
        jnp.arange(k2, dtype=jnp.int32))
    em2 = em1 & keep1[src1] & keep1[dst1]

    dst_l2 = padi(jnp.where(em2, inv1[dst1], k2), k2)
    a16_2p = _sc_attr_deg(vals, dst_l2, nacc2)
    a16_2 = a16_2p[0] + a16_2p[1]
    r2 = lax.rsqrt(1.0 + a16_2[:k2, 4])

    sfull2 = jnp.zeros((k1,), jnp.float32).at[perm1].set(sig1 * r2)
    src_l1 = padi(src1, k1)
    h_b = _conv_epilogue(
        _sc_gather_scatter(h_d1 * sfull2[:, None], src_l1, dst_l2, nacc2),
        a16_2, P['We_b'], P['W_b'], P['b_b'], k2)

    su0 = r1[perm1]
    src_u0 = padi(jnp.where(keep1[src1], inv1[src1], k2), k2)
    h_u0 = _conv_epilogue(
        _sc_gather_scatter(h_b * su0[:, None], src_u0, dst_l1, nacc1),
        a16_1, P['We_u0'], P['W_u0'], P['b_u0'], k1, skip=h_d1)

    su1 = r0[perm0]
    src_u1 = padi(jnp.where(keep0[src0], inv0[src0], k1), k1)
    h_u1 = _conv_epilogue(
        _sc_gather_scatter(h_u0 * su1[:, None], src_u1, dst_l0, nacc0),
        a16_0, P['We_u1'], P['W_u1'], P['b_u1'], n0, skip=h_d0)

    h_fin = h_u1 + feat

    rep = (_segment_mean(h_u0, bat1, g) + _segment_mean(h_u1, batch, g)
           + _segment_mean(h_fin, batch, g))
    return _mm(rep, P['W_pred'], P['b_pred'], relu=False)
